# baseline (speedup 1.0000x reference)
import jax
import jax.numpy as jnp
from jax.experimental import pallas as pl
from jax.experimental.pallas import tpu as pltpu

_EPS = 1e-05
_BLOCK = 16384
_CHUNK = 2048


def _make_body(n):
    nb_full = n // _BLOCK
    tail = n - nb_full * _BLOCK
    ni = nb_full + (1 if tail else 0)

    def body(g_hbm, q_hbm, ixoz_ref, mxoz_ref, ixoy_ref, mxoy_ref,
             iyoz_ref, myoz_ref, gbuf, qbuf, sems):
        i = pl.program_id(0)
        slot = jax.lax.rem(i, 2)
        nxt = jax.lax.rem(i + 1, 2)

        def full_copies(k, buf_slot):
            return (
                pltpu.make_async_copy(
                    g_hbm.at[pl.ds(k * _BLOCK, _BLOCK), :],
                    gbuf.at[buf_slot], sems.at[0, buf_slot]),
                pltpu.make_async_copy(
                    q_hbm.at[pl.ds(k * _BLOCK, _BLOCK), :],
                    qbuf.at[buf_slot], sems.at[1, buf_slot]),
            )

        def tail_copies(buf_slot):
            return (
                pltpu.make_async_copy(
                    g_hbm.at[pl.ds(nb_full * _BLOCK, tail), :],
                    gbuf.at[buf_slot, 0:tail, :], sems.at[0, buf_slot]),
                pltpu.make_async_copy(
                    q_hbm.at[pl.ds(nb_full * _BLOCK, tail), :],
                    qbuf.at[buf_slot, 0:tail, :], sems.at[1, buf_slot]),
            )

        def start_block(k, buf_slot):
            @pl.when(k < nb_full)
            def _():
                for cp in full_copies(k, buf_slot):
                    cp.start()
            if tail:
                @pl.when(k == nb_full)
                def _():
                    for cp in tail_copies(buf_slot):
                        cp.start()

        @pl.when(i == 0)
        def _():
            start_block(i, slot)

        @pl.when(i + 1 < ni)
        def _():
            start_block(i + 1, nxt)

        @pl.when(i < nb_full)
        def _():
            for cp in full_copies(i, slot):
                cp.wait()
        if tail:
            @pl.when(i == nb_full)
            def _():
                for cp in tail_copies(slot):
                    cp.wait()

        for c in range(_BLOCK // _CHUNK):
            lo = c * _CHUNK
            gt = gbuf[slot, lo:lo + _CHUNK, :].T
            qt = qbuf[slot, lo:lo + _CHUNK, :].T
            glo = gt[0:3, :] - 0.5 * gt[3:6, :]
            ghi = gt[0:3, :] + 0.5 * gt[3:6, :]
            qlo = qt[0:3, :] - 0.5 * qt[3:6, :]
            qhi = qt[0:3, :] + 0.5 * qt[3:6, :]
            iw = jnp.minimum(ghi, qhi) - jnp.maximum(glo, qlo) + _EPS
            mw = jnp.maximum(ghi, qhi) - jnp.minimum(glo, qlo) + _EPS

            def _plane(a, b, i_ref, m_ref):
                wa, wb = iw[a:a + 1, :], iw[b:b + 1, :]
                inter = jnp.where((wa > 0.0) & (wb > 0.0), wa * wb, 0.0)
                mbr = mw[a:a + 1, :] * mw[b:b + 1, :]
                i_ref[lo:lo + _CHUNK] = inter.reshape(_CHUNK)
                m_ref[lo:lo + _CHUNK] = mbr.reshape(_CHUNK)

            _plane(0, 2, ixoz_ref, mxoz_ref)
            _plane(0, 1, ixoy_ref, mxoy_ref)
            _plane(1, 2, iyoz_ref, myoz_ref)

    return body, ni


def kernel(gboxes, qboxes):
    n = gboxes.shape[0]
    body, ni = _make_body(n)
    in_spec = pl.BlockSpec(memory_space=pl.ANY)
    out_spec = pl.BlockSpec((_BLOCK,), lambda i: (i,))
    out_shape = tuple(jax.ShapeDtypeStruct((n,), jnp.float32)
                      for _ in range(6))
    return pl.pallas_call(
        body,
        out_shape=out_shape,
        grid=(ni,),
        in_specs=[in_spec, in_spec],
        out_specs=[out_spec] * 6,
        scratch_shapes=[
            pltpu.VMEM((2, _BLOCK, 7), jnp.float32),
            pltpu.VMEM((2, _BLOCK, 7), jnp.float32),
            pltpu.SemaphoreType.DMA((2, 2)),
        ],
        compiler_params=pltpu.CompilerParams(
            dimension_semantics=("arbitrary",),
            vmem_limit_bytes=60 * 1024 * 1024,
        ),
        name="align_inter_aligned",
    )(gboxes, qboxes)


# speedup vs baseline: 3.9086x; 3.9086x over previous
import jax
import jax.numpy as jnp
from jax.experimental import pallas as pl
from jax.experimental.pallas import tpu as pltpu

_EPS = 1e-05
_BS = 1024


def _align_body(g_ref, q_ref, ixoz_ref, mxoz_ref, ixoy_ref, mxoy_ref,
                iyoz_ref, myoz_ref):
    g3 = g_ref[...]
    q3 = q_ref[...]
    glo = g3[0:3] - 0.5 * g3[3:6]
    ghi = g3[0:3] + 0.5 * g3[3:6]
    qlo = q3[0:3] - 0.5 * q3[3:6]
    qhi = q3[0:3] + 0.5 * q3[3:6]
    iw = jnp.minimum(ghi, qhi) - jnp.maximum(glo, qlo) + _EPS
    mw = jnp.maximum(ghi, qhi) - jnp.minimum(glo, qlo) + _EPS
    pos = iw > 0.0

    def _plane(a, b, i_ref, m_ref):
        inter = jnp.where(pos[a] & pos[b], iw[a] * iw[b], 0.0)
        i_ref[...] = inter
        m_ref[...] = mw[a] * mw[b]

    _plane(0, 2, ixoz_ref, mxoz_ref)
    _plane(0, 1, ixoy_ref, mxoy_ref)
    _plane(1, 2, iyoz_ref, myoz_ref)


def kernel(gboxes, qboxes):
    n = gboxes.shape[0]
    m = n // 128
    gt = jnp.moveaxis(gboxes, 1, 0).reshape(7, m, 128)
    qt = jnp.moveaxis(qboxes, 1, 0).reshape(7, m, 128)
    grid = (pl.cdiv(m, _BS),)
    in_spec = pl.BlockSpec((7, _BS, 128), lambda i: (0, i, 0))
    out_spec = pl.BlockSpec((_BS, 128), lambda i: (i, 0))
    out_shape = tuple(jax.ShapeDtypeStruct((m, 128), jnp.float32)
                      for _ in range(6))
    outs = pl.pallas_call(
        _align_body,
        out_shape=out_shape,
        grid=grid,
        in_specs=[in_spec, in_spec],
        out_specs=[out_spec] * 6,
        compiler_params=pltpu.CompilerParams(
            dimension_semantics=("arbitrary",),
        ),
        name="align_inter_aligned",
    )(gt, qt)
    return tuple(o.reshape(n) for o in outs)


# speedup vs baseline: 16.3478x; 4.1825x over previous
import jax
import jax.numpy as jnp
from jax.experimental import pallas as pl
from jax.experimental.pallas import tpu as pltpu

_EPS = 1e-05
_BL = 65536


def _align_body(g_ref, q_ref, ixoz_ref, mxoz_ref, ixoy_ref, mxoy_ref,
                iyoz_ref, myoz_ref):
    g = g_ref[...]
    q = q_ref[...]
    glo = g[0:3] - 0.5 * g[3:6]
    ghi = g[0:3] + 0.5 * g[3:6]
    qlo = q[0:3] - 0.5 * q[3:6]
    qhi = q[0:3] + 0.5 * q[3:6]
    iw = jnp.minimum(ghi, qhi) - jnp.maximum(glo, qlo) + _EPS
    mw = jnp.maximum(ghi, qhi) - jnp.minimum(glo, qlo) + _EPS
    pos = iw > 0.0

    def _plane(a, b, i_ref, m_ref):
        inter = jnp.where(pos[a:a + 1] & pos[b:b + 1],
                          iw[a:a + 1] * iw[b:b + 1], 0.0)
        i_ref[...] = inter.reshape(_BL)
        m_ref[...] = (mw[a:a + 1] * mw[b:b + 1]).reshape(_BL)

    _plane(0, 2, ixoz_ref, mxoz_ref)
    _plane(0, 1, ixoy_ref, mxoy_ref)
    _plane(1, 2, iyoz_ref, myoz_ref)


def kernel(gboxes, qboxes):
    n = gboxes.shape[0]
    gt = gboxes.T
    qt = qboxes.T
    grid = (pl.cdiv(n, _BL),)
    in_spec = pl.BlockSpec((7, _BL), lambda i: (0, i))
    out_spec = pl.BlockSpec((_BL,), lambda i: (i,))
    out_shape = tuple(jax.ShapeDtypeStruct((n,), jnp.float32)
                      for _ in range(6))
    return pl.pallas_call(
        _align_body,
        out_shape=out_shape,
        grid=grid,
        in_specs=[in_spec, in_spec],
        out_specs=[out_spec] * 6,
        compiler_params=pltpu.CompilerParams(
            dimension_semantics=("arbitrary",),
        ),
        name="align_inter_aligned",
    )(gt, qt)


# speedup vs baseline: 29.8113x; 1.8236x over previous
import jax
import jax.numpy as jnp
from jax.experimental import pallas as pl
from jax.experimental.pallas import tpu as pltpu

_EPS = 1e-05
_BL = 65536
_COLS = 6


def _compute(cols):
    g, gd, q, qd = cols[0:3], cols[3:6], cols[6:9], cols[9:12]
    iw, mw, pos = [], [], []
    for a in range(3):
        ghalf = 0.5 * gd[a]
        qhalf = 0.5 * qd[a]
        glo, ghi = g[a] - ghalf, g[a] + ghalf
        qlo, qhi = q[a] - qhalf, q[a] + qhalf
        w = jnp.minimum(ghi, qhi) - jnp.maximum(glo, qlo) + _EPS
        iw.append(w)
        mw.append(jnp.maximum(ghi, qhi) - jnp.minimum(glo, qlo) + _EPS)
        pos.append(w > 0.0)

    outs = []
    for a, b in ((0, 2), (0, 1), (1, 2)):
        outs.append(jnp.where(pos[a] & pos[b], iw[a] * iw[b], 0.0))
        outs.append(mw[a] * mw[b])
    return outs


def _make_body(n):
    nb_full = n // _BL
    tail = n - nb_full * _BL
    ni = nb_full + (1 if tail else 0)

    def body(g_hbm, q_hbm, *refs):
        out_refs = refs[0:6]
        gbuf, qbuf, sems = refs[6], refs[7], refs[8]
        tail_bufs = refs[9:9 + (2 * _COLS if tail else 0)]
        tail_sems = refs[9 + 2 * _COLS] if tail else None
        i = pl.program_id(0)
        slot = jax.lax.rem(i, 2)
        nxt = jax.lax.rem(i + 1, 2)

        def full_copies(k, buf_slot):
            cps = []
            for c in range(_COLS):
                cps.append(pltpu.make_async_copy(
                    g_hbm.at[c, pl.ds(k * _BL, _BL)],
                    gbuf.at[pl.ds((buf_slot * _COLS + c) * _BL, _BL)],
                    sems.at[0, buf_slot, c]))
                cps.append(pltpu.make_async_copy(
                    q_hbm.at[c, pl.ds(k * _BL, _BL)],
                    qbuf.at[pl.ds((buf_slot * _COLS + c) * _BL, _BL)],
                    sems.at[1, buf_slot, c]))
            return cps

        def tail_copies():
            cps = []
            for c in range(_COLS):
                cps.append(pltpu.make_async_copy(
                    g_hbm.at[c, pl.ds(nb_full * _BL, tail)],
                    tail_bufs[c], tail_sems.at[0, c]))
                cps.append(pltpu.make_async_copy(
                    q_hbm.at[c, pl.ds(nb_full * _BL, tail)],
                    tail_bufs[_COLS + c], tail_sems.at[1, c]))
            return cps

        def start_block(k, buf_slot):
            @pl.when(k < nb_full)
            def _():
                for cp in full_copies(k, buf_slot):
                    cp.start()
            if tail:
                @pl.when(k == nb_full)
                def _():
                    for cp in tail_copies():
                        cp.start()

        @pl.when(i == 0)
        def _():
            start_block(i, slot)

        @pl.when(i + 1 < ni)
        def _():
            start_block(i + 1, nxt)

        @pl.when(i < nb_full)
        def _():
            for cp in full_copies(i, slot):
                cp.wait()
            cols = [gbuf[pl.ds((slot * _COLS + c) * _BL, _BL)]
                    for c in range(_COLS)]
            cols += [qbuf[pl.ds((slot * _COLS + c) * _BL, _BL)]
                     for c in range(_COLS)]
            for o_ref, val in zip(out_refs, _compute(cols)):
                o_ref[...] = val

        if tail:
            @pl.when(i == nb_full)
            def _():
                for cp in tail_copies():
                    cp.wait()
                cols = [tail_bufs[j][...] for j in range(2 * _COLS)]
                pad = jnp.zeros((_BL - tail,), jnp.float32)
                for o_ref, val in zip(out_refs, _compute(cols)):
                    o_ref[...] = jnp.concatenate([val, pad])

    scratch = [
        pltpu.VMEM((2 * _COLS * _BL,), jnp.float32),
        pltpu.VMEM((2 * _COLS * _BL,), jnp.float32),
        pltpu.SemaphoreType.DMA((2, 2, _COLS)),
    ]
    if tail:
        scratch += [pltpu.VMEM((tail,), jnp.float32)] * (2 * _COLS)
        scratch += [pltpu.SemaphoreType.DMA((2, _COLS))]
    return body, ni, scratch


def kernel(gboxes, qboxes):
    n = gboxes.shape[0]
    gt = gboxes.T
    qt = qboxes.T
    body, ni, scratch = _make_body(n)
    in_spec = pl.BlockSpec(memory_space=pl.ANY)
    out_spec = pl.BlockSpec((_BL,), lambda i: (i,))
    out_shape = tuple(jax.ShapeDtypeStruct((n,), jnp.float32)
                      for _ in range(6))
    return pl.pallas_call(
        body,
        out_shape=out_shape,
        grid=(ni,),
        in_specs=[in_spec, in_spec],
        out_specs=[out_spec] * 6,
        scratch_shapes=scratch,
        compiler_params=pltpu.CompilerParams(
            dimension_semantics=("arbitrary",),
        ),
        name="align_inter_aligned",
    )(gt, qt)


# speedup vs baseline: 33.0381x; 1.1082x over previous
import jax
import jax.numpy as jnp
from jax.experimental import pallas as pl
from jax.experimental.pallas import tpu as pltpu

_EPS = 1e-05
_BL = 131072
_COLS = 6


def _compute(cols):
    g, gd, q, qd = cols[0:3], cols[3:6], cols[6:9], cols[9:12]
    iw, mw, pos = [], [], []
    for a in range(3):
        ghalf = 0.5 * gd[a]
        qhalf = 0.5 * qd[a]
        glo, ghi = g[a] - ghalf, g[a] + ghalf
        qlo, qhi = q[a] - qhalf, q[a] + qhalf
        w = jnp.minimum(ghi, qhi) - jnp.maximum(glo, qlo) + _EPS
        iw.append(w)
        mw.append(jnp.maximum(ghi, qhi) - jnp.minimum(glo, qlo) + _EPS)
        pos.append(w > 0.0)

    outs = []
    for a, b in ((0, 2), (0, 1), (1, 2)):
        outs.append(jnp.where(pos[a] & pos[b], iw[a] * iw[b], 0.0))
        outs.append(mw[a] * mw[b])
    return outs


def _make_body(n):
    nb_full = n // _BL
    tail = n - nb_full * _BL
    ni = nb_full + (1 if tail else 0)

    def body(g_hbm, q_hbm, *refs):
        out_refs = refs[0:6]
        gbuf, qbuf, sems = refs[6], refs[7], refs[8]
        tail_bufs = refs[9:9 + (2 * _COLS if tail else 0)]
        tail_sems = refs[9 + 2 * _COLS] if tail else None
        i = pl.program_id(0)
        slot = jax.lax.rem(i, 2)
        nxt = jax.lax.rem(i + 1, 2)

        def full_copies(k, buf_slot):
            cps = []
            for c in range(_COLS):
                cps.append(pltpu.make_async_copy(
                    g_hbm.at[c, pl.ds(k * _BL, _BL)],
                    gbuf.at[pl.ds((buf_slot * _COLS + c) * _BL, _BL)],
                    sems.at[0, buf_slot, c]))
                cps.append(pltpu.make_async_copy(
                    q_hbm.at[c, pl.ds(k * _BL, _BL)],
                    qbuf.at[pl.ds((buf_slot * _COLS + c) * _BL, _BL)],
                    sems.at[1, buf_slot, c]))
            return cps

        def tail_copies():
            cps = []
            for c in range(_COLS):
                cps.append(pltpu.make_async_copy(
                    g_hbm.at[c, pl.ds(nb_full * _BL, tail)],
                    tail_bufs[c], tail_sems.at[0, c]))
                cps.append(pltpu.make_async_copy(
                    q_hbm.at[c, pl.ds(nb_full * _BL, tail)],
                    tail_bufs[_COLS + c], tail_sems.at[1, c]))
            return cps

        def start_block(k, buf_slot):
            @pl.when(k < nb_full)
            def _():
                for cp in full_copies(k, buf_slot):
                    cp.start()
            if tail:
                @pl.when(k == nb_full)
                def _():
                    for cp in tail_copies():
                        cp.start()

        @pl.when(i == 0)
        def _():
            start_block(i, slot)

        @pl.when(i + 1 < ni)
        def _():
            start_block(i + 1, nxt)

        @pl.when(i < nb_full)
        def _():
            for cp in full_copies(i, slot):
                cp.wait()
            cols = [gbuf[pl.ds((slot * _COLS + c) * _BL, _BL)]
                    for c in range(_COLS)]
            cols += [qbuf[pl.ds((slot * _COLS + c) * _BL, _BL)]
                     for c in range(_COLS)]
            for o_ref, val in zip(out_refs, _compute(cols)):
                o_ref[...] = val

        if tail:
            @pl.when(i == nb_full)
            def _():
                for cp in tail_copies():
                    cp.wait()
                cols = [tail_bufs[j][...] for j in range(2 * _COLS)]
                pad = jnp.zeros((_BL - tail,), jnp.float32)
                for o_ref, val in zip(out_refs, _compute(cols)):
                    o_ref[...] = jnp.concatenate([val, pad])

    scratch = [
        pltpu.VMEM((2 * _COLS * _BL,), jnp.float32),
        pltpu.VMEM((2 * _COLS * _BL,), jnp.float32),
        pltpu.SemaphoreType.DMA((2, 2, _COLS)),
    ]
    if tail:
        scratch += [pltpu.VMEM((tail,), jnp.float32)] * (2 * _COLS)
        scratch += [pltpu.SemaphoreType.DMA((2, _COLS))]
    return body, ni, scratch


def kernel(gboxes, qboxes):
    n = gboxes.shape[0]
    gt = gboxes.T
    qt = qboxes.T
    body, ni, scratch = _make_body(n)
    in_spec = pl.BlockSpec(memory_space=pl.ANY)
    out_spec = pl.BlockSpec((_BL,), lambda i: (i,))
    out_shape = tuple(jax.ShapeDtypeStruct((n,), jnp.float32)
                      for _ in range(6))
    return pl.pallas_call(
        body,
        out_shape=out_shape,
        grid=(ni,),
        in_specs=[in_spec, in_spec],
        out_specs=[out_spec] * 6,
        scratch_shapes=scratch,
        compiler_params=pltpu.CompilerParams(
            dimension_semantics=("arbitrary",),
        ),
        name="align_inter_aligned",
    )(gt, qt)


# speedup vs baseline: 34.0814x; 1.0316x over previous
import jax
import jax.numpy as jnp
from jax.experimental import pallas as pl
from jax.experimental.pallas import tpu as pltpu

_EPS = 1e-05
_BL = 196608
_COLS = 6


def _compute(cols):
    g, gd, q, qd = cols[0:3], cols[3:6], cols[6:9], cols[9:12]
    iw, mw, pos = [], [], []
    for a in range(3):
        ghalf = 0.5 * gd[a]
        qhalf = 0.5 * qd[a]
        glo, ghi = g[a] - ghalf, g[a] + ghalf
        qlo, qhi = q[a] - qhalf, q[a] + qhalf
        w = jnp.minimum(ghi, qhi) - jnp.maximum(glo, qlo) + _EPS
        iw.append(w)
        mw.append(jnp.maximum(ghi, qhi) - jnp.minimum(glo, qlo) + _EPS)
        pos.append(w > 0.0)

    outs = []
    for a, b in ((0, 2), (0, 1), (1, 2)):
        outs.append(jnp.where(pos[a] & pos[b], iw[a] * iw[b], 0.0))
        outs.append(mw[a] * mw[b])
    return outs


def _make_body(n):
    nb_full = n // _BL
    tail = n - nb_full * _BL
    ni = nb_full + (1 if tail else 0)

    def body(g_hbm, q_hbm, *refs):
        out_refs = refs[0:6]
        gbuf, qbuf, sems = refs[6], refs[7], refs[8]
        tail_bufs = refs[9:9 + (2 * _COLS if tail else 0)]
        tail_sems = refs[9 + 2 * _COLS] if tail else None
        i = pl.program_id(0)
        slot = jax.lax.rem(i, 2)
        nxt = jax.lax.rem(i + 1, 2)

        def full_copies(k, buf_slot):
            cps = []
            for c in range(_COLS):
                cps.append(pltpu.make_async_copy(
                    g_hbm.at[c, pl.ds(k * _BL, _BL)],
                    gbuf.at[pl.ds((buf_slot * _COLS + c) * _BL, _BL)],
                    sems.at[0, buf_slot, c]))
                cps.append(pltpu.make_async_copy(
                    q_hbm.at[c, pl.ds(k * _BL, _BL)],
                    qbuf.at[pl.ds((buf_slot * _COLS + c) * _BL, _BL)],
                    sems.at[1, buf_slot, c]))
            return cps

        def tail_copies():
            cps = []
            for c in range(_COLS):
                cps.append(pltpu.make_async_copy(
                    g_hbm.at[c, pl.ds(nb_full * _BL, tail)],
                    tail_bufs[c], tail_sems.at[0, c]))
                cps.append(pltpu.make_async_copy(
                    q_hbm.at[c, pl.ds(nb_full * _BL, tail)],
                    tail_bufs[_COLS + c], tail_sems.at[1, c]))
            return cps

        def start_block(k, buf_slot):
            @pl.when(k < nb_full)
            def _():
                for cp in full_copies(k, buf_slot):
                    cp.start()
            if tail:
                @pl.when(k == nb_full)
                def _():
                    for cp in tail_copies():
                        cp.start()

        @pl.when(i == 0)
        def _():
            start_block(i, slot)

        @pl.when(i + 1 < ni)
        def _():
            start_block(i + 1, nxt)

        @pl.when(i < nb_full)
        def _():
            for cp in full_copies(i, slot):
                cp.wait()
            cols = [gbuf[pl.ds((slot * _COLS + c) * _BL, _BL)]
                    for c in range(_COLS)]
            cols += [qbuf[pl.ds((slot * _COLS + c) * _BL, _BL)]
                     for c in range(_COLS)]
            for o_ref, val in zip(out_refs, _compute(cols)):
                o_ref[...] = val

        if tail:
            @pl.when(i == nb_full)
            def _():
                for cp in tail_copies():
                    cp.wait()
                cols = [tail_bufs[j][...] for j in range(2 * _COLS)]
                pad = jnp.zeros((_BL - tail,), jnp.float32)
                for o_ref, val in zip(out_refs, _compute(cols)):
                    o_ref[...] = jnp.concatenate([val, pad])

    scratch = [
        pltpu.VMEM((2 * _COLS * _BL,), jnp.float32),
        pltpu.VMEM((2 * _COLS * _BL,), jnp.float32),
        pltpu.SemaphoreType.DMA((2, 2, _COLS)),
    ]
    if tail:
        scratch += [pltpu.VMEM((tail,), jnp.float32)] * (2 * _COLS)
        scratch += [pltpu.SemaphoreType.DMA((2, _COLS))]
    return body, ni, scratch


def kernel(gboxes, qboxes):
    n = gboxes.shape[0]
    gt = gboxes.T
    qt = qboxes.T
    body, ni, scratch = _make_body(n)
    in_spec = pl.BlockSpec(memory_space=pl.ANY)
    out_spec = pl.BlockSpec((_BL,), lambda i: (i,))
    out_shape = tuple(jax.ShapeDtypeStruct((n,), jnp.float32)
                      for _ in range(6))
    return pl.pallas_call(
        body,
        out_shape=out_shape,
        grid=(ni,),
        in_specs=[in_spec, in_spec],
        out_specs=[out_spec] * 6,
        scratch_shapes=scratch,
        compiler_params=pltpu.CompilerParams(
            dimension_semantics=("arbitrary",),
            vmem_limit_bytes=56 * 1024 * 1024,
        ),
        name="align_inter_aligned",
    )(gt, qt)


# speedup vs baseline: 34.5442x; 1.0136x over previous
import jax
import jax.numpy as jnp
from jax.experimental import pallas as pl
from jax.experimental.pallas import tpu as pltpu

_EPS = 1e-05
_BL = 262144
_COLS = 6


def _compute(cols):
    g, gd, q, qd = cols[0:3], cols[3:6], cols[6:9], cols[9:12]
    iw, mw, pos = [], [], []
    for a in range(3):
        ghalf = 0.5 * gd[a]
        qhalf = 0.5 * qd[a]
        glo, ghi = g[a] - ghalf, g[a] + ghalf
        qlo, qhi = q[a] - qhalf, q[a] + qhalf
        w = jnp.minimum(ghi, qhi) - jnp.maximum(glo, qlo) + _EPS
        iw.append(w)
        mw.append(jnp.maximum(ghi, qhi) - jnp.minimum(glo, qlo) + _EPS)
        pos.append(w > 0.0)

    outs = []
    for a, b in ((0, 2), (0, 1), (1, 2)):
        outs.append(jnp.where(pos[a] & pos[b], iw[a] * iw[b], 0.0))
        outs.append(mw[a] * mw[b])
    return outs


def _make_body(n):
    nb_full = n // _BL
    tail = n - nb_full * _BL
    ni = nb_full + (1 if tail else 0)

    def body(g_hbm, q_hbm, *refs):
        out_refs = refs[0:6]
        gbuf, qbuf, sems = refs[6], refs[7], refs[8]
        tail_bufs = refs[9:9 + (2 * _COLS if tail else 0)]
        tail_sems = refs[9 + 2 * _COLS] if tail else None
        i = pl.program_id(0)
        slot = jax.lax.rem(i, 2)
        nxt = jax.lax.rem(i + 1, 2)

        def full_copies(k, buf_slot):
            cps = []
            for c in range(_COLS):
                cps.append(pltpu.make_async_copy(
                    g_hbm.at[c, pl.ds(k * _BL, _BL)],
                    gbuf.at[pl.ds((buf_slot * _COLS + c) * _BL, _BL)],
                    sems.at[0, buf_slot, c]))
                cps.append(pltpu.make_async_copy(
                    q_hbm.at[c, pl.ds(k * _BL, _BL)],
                    qbuf.at[pl.ds((buf_slot * _COLS + c) * _BL, _BL)],
                    sems.at[1, buf_slot, c]))
            return cps

        def tail_copies():
            cps = []
            for c in range(_COLS):
                cps.append(pltpu.make_async_copy(
                    g_hbm.at[c, pl.ds(nb_full * _BL, tail)],
                    tail_bufs[c], tail_sems.at[0, c]))
                cps.append(pltpu.make_async_copy(
                    q_hbm.at[c, pl.ds(nb_full * _BL, tail)],
                    tail_bufs[_COLS + c], tail_sems.at[1, c]))
            return cps

        def start_block(k, buf_slot):
            @pl.when(k < nb_full)
            def _():
                for cp in full_copies(k, buf_slot):
                    cp.start()
            if tail:
                @pl.when(k == nb_full)
                def _():
                    for cp in tail_copies():
                        cp.start()

        @pl.when(i == 0)
        def _():
            start_block(i, slot)

        @pl.when(i + 1 < ni)
        def _():
            start_block(i + 1, nxt)

        @pl.when(i < nb_full)
        def _():
            for cp in full_copies(i, slot):
                cp.wait()
            cols = [gbuf[pl.ds((slot * _COLS + c) * _BL, _BL)]
                    for c in range(_COLS)]
            cols += [qbuf[pl.ds((slot * _COLS + c) * _BL, _BL)]
                     for c in range(_COLS)]
            for o_ref, val in zip(out_refs, _compute(cols)):
                o_ref[...] = val

        if tail:
            @pl.when(i == nb_full)
            def _():
                for cp in tail_copies():
                    cp.wait()
                cols = [tail_bufs[j][...] for j in range(2 * _COLS)]
                pad = jnp.zeros((_BL - tail,), jnp.float32)
                for o_ref, val in zip(out_refs, _compute(cols)):
                    o_ref[...] = jnp.concatenate([val, pad])

    scratch = [
        pltpu.VMEM((2 * _COLS * _BL,), jnp.float32),
        pltpu.VMEM((2 * _COLS * _BL,), jnp.float32),
        pltpu.SemaphoreType.DMA((2, 2, _COLS)),
    ]
    if tail:
        scratch += [pltpu.VMEM((tail,), jnp.float32)] * (2 * _COLS)
        scratch += [pltpu.SemaphoreType.DMA((2, _COLS))]
    return body, ni, scratch


def kernel(gboxes, qboxes):
    n = gboxes.shape[0]
    gt = gboxes.T
    qt = qboxes.T
    body, ni, scratch = _make_body(n)
    in_spec = pl.BlockSpec(memory_space=pl.ANY)
    out_spec = pl.BlockSpec((_BL,), lambda i: (i,))
    out_shape = tuple(jax.ShapeDtypeStruct((n,), jnp.float32)
                      for _ in range(6))
    return pl.pallas_call(
        body,
        out_shape=out_shape,
        grid=(ni,),
        in_specs=[in_spec, in_spec],
        out_specs=[out_spec] * 6,
        scratch_shapes=scratch,
        compiler_params=pltpu.CompilerParams(
            dimension_semantics=("arbitrary",),
            vmem_limit_bytes=56 * 1024 * 1024,
        ),
        name="align_inter_aligned",
    )(gt, qt)
